# Initial kernel scaffold; baseline (speedup 1.0000x reference)
import jax
import jax.numpy as jnp
from jax.experimental import pallas as pl


def kernel(input_xyz, num_pred, params):
    raise NotImplementedError("write your pallas kernel here")


# speedup vs baseline: 3.7613x; 3.7613x over previous
import functools

import jax
import jax.numpy as jnp
import numpy as np
from jax.experimental import pallas as pl
from jax.experimental.pallas import tpu as pltpu

_F32 = jnp.float32
_BIG = 3.0e38
_PREC = jax.lax.Precision.HIGHEST


def _dot(a, b):
    return jax.lax.dot_general(a, b, (((1,), (0,)), ((), ())),
                               preferred_element_type=_F32, precision=_PREC)


def _first_min_onehot(D, lane_iota, n):
    mn = jnp.min(D, axis=1, keepdims=True)
    idx = jnp.min(jnp.where(D == mn, lane_iota, n), axis=1, keepdims=True)
    oh = (lane_iota == idx).astype(_F32)
    return oh, mn, idx


def _sa_kernel(featxyz_ref, xyz_ref, xyzT_ref,
               w1_ref, b1_ref, w2_ref, b2_ref, w3_ref, b3_ref,
               fout_ref, cen_ref, d_ref,
               *, n, m, k, cf):
    xyzT = xyzT_ref[...]
    lane_n = jax.lax.broadcasted_iota(jnp.int32, (1, n), 1)

    d0 = jnp.sum((xyzT - xyzT[:, 0:1]) ** 2, axis=0, keepdims=True)
    d_ref[0:1, :] = d0
    cen_ref[0:1, :] = xyz_ref[0:1, :]

    def fps_body(i, dists):
        mx = jnp.max(dists)
        idx = jnp.min(jnp.where(dists == mx, lane_n, n))
        cen_ref[pl.ds(i, 1), :] = xyz_ref[pl.ds(idx, 1), :]
        mask = (lane_n == idx).astype(_F32)
        col = jnp.sum(xyzT * mask, axis=1, keepdims=True)
        dnew = jnp.sum((xyzT - col) ** 2, axis=0, keepdims=True)
        d_ref[pl.ds(i, 1), :] = dnew
        return jnp.minimum(dists, dnew)

    jax.lax.fori_loop(1, m, fps_body, d0)

    lane_mn = jax.lax.broadcasted_iota(jnp.int32, (m, n), 1)
    cen = cen_ref[...]
    cenpad = jnp.concatenate([jnp.zeros((m, cf), _F32), cen], axis=1)
    featxyz = featxyz_ref[...]
    w1, b1 = w1_ref[...], b1_ref[...]
    w2, b2 = w2_ref[...], b2_ref[...]
    w3, b3 = w3_ref[...], b3_ref[...]

    def nbr_body(_, acc):
        D = d_ref[...]
        oh, _, idx = _first_min_onehot(D, lane_mn, n)
        d_ref[...] = jnp.where(lane_mn == idx, _BIG, D)
        g = _dot(oh, featxyz) - cenpad
        a = jnp.maximum(_dot(g, w1) + b1, 0.0)
        a = jnp.maximum(_dot(a, w2) + b2, 0.0)
        a = jnp.maximum(_dot(a, w3) + b3, 0.0)
        return jnp.maximum(acc, a)

    cout = w3.shape[1]
    fout_ref[...] = jax.lax.fori_loop(
        0, k, nbr_body, jnp.zeros((m, cout), _F32))


def _sa(layers, feat, xyz, m, k):
    B, n, cf = feat.shape
    (w1, b1), (w2, b2), (w3, b3) = layers
    cout = w3.shape[1]
    featxyz = jnp.concatenate([feat, xyz], axis=2)
    xyzT = jnp.transpose(xyz, (0, 2, 1))
    fn = pl.pallas_call(
        functools.partial(_sa_kernel, n=n, m=m, k=k, cf=cf),
        out_shape=(jax.ShapeDtypeStruct((m, cout), _F32),
                   jax.ShapeDtypeStruct((m, 3), _F32)),
        scratch_shapes=[pltpu.VMEM((m, n), _F32)],
    )
    fout, cen = jax.vmap(fn, in_axes=(0, 0, 0) + (None,) * 6)(
        featxyz, xyz, xyzT,
        w1, b1.reshape(1, -1), w2, b2.reshape(1, -1), w3, b3.reshape(1, -1))
    return fout, cen


def _lpt_kernel(fcur_ref, fsrc_ref, qxyz_ref, sxyzT_ref,
                wq_ref, wk_ref, wv_ref, out_ref, d_ref, s_ref, v_ref,
                *, m, n, k, c):
    qxyz = qxyz_ref[...]
    sxyzT = sxyzT_ref[...]
    D = ((qxyz[:, 0:1] - sxyzT[0:1, :]) ** 2
         + (qxyz[:, 1:2] - sxyzT[1:2, :]) ** 2)
    d_ref[...] = D + (qxyz[:, 2:3] - sxyzT[2:3, :]) ** 2
    lane_mn = jax.lax.broadcasted_iota(jnp.int32, (m, n), 1)

    q = _dot(fcur_ref[...], wq_ref[...])
    K = _dot(fsrc_ref[...], wk_ref[...])
    V = _dot(fsrc_ref[...], wv_ref[...])

    def gather_body(j, _):
        D = d_ref[...]
        oh, _, idx = _first_min_onehot(D, lane_mn, n)
        d_ref[...] = jnp.where(lane_mn == idx, _BIG, D)
        kj = _dot(oh, K)
        s_ref[j] = jnp.sum(kj * q, axis=1, keepdims=True)
        v_ref[j] = _dot(oh, V)
        return 0

    jax.lax.fori_loop(0, k, gather_body, 0)

    s = s_ref[...] / np.sqrt(c)
    e = jnp.exp(s - jnp.max(s, axis=0, keepdims=True))
    s_ref[...] = e / jnp.sum(e, axis=0, keepdims=True)

    def mix_body(j, out):
        return out + s_ref[j] * v_ref[j]

    out_ref[...] = jax.lax.fori_loop(
        0, k, mix_body, jnp.zeros((m, c), _F32))


def _lpt(p, f_cur, f_src, xyz_cur, xyz_src, k):
    B, m, c = f_cur.shape
    n = f_src.shape[1]
    sxyzT = jnp.transpose(xyz_src, (0, 2, 1))
    fn = pl.pallas_call(
        functools.partial(_lpt_kernel, m=m, n=n, k=k, c=c),
        out_shape=jax.ShapeDtypeStruct((m, c), _F32),
        scratch_shapes=[pltpu.VMEM((m, n), _F32),
                        pltpu.VMEM((k, m, 1), _F32),
                        pltpu.VMEM((k, m, c), _F32)],
    )
    return jax.vmap(fn, in_axes=(0, 0, 0, 0, None, None, None))(
        f_cur, f_src, xyz_cur, sxyzT, p['Wq'], p['Wk'], p['Wv'])


def _lstm_kernel(fb_ref, ff_ref, h_ref, c_ref, wx_ref, wh_ref, b_ref,
                 hout_ref, cout_ref, *, hdim):
    x = jnp.concatenate([fb_ref[...], ff_ref[...]], axis=1)
    g = _dot(x, wx_ref[...]) + _dot(h_ref[...], wh_ref[...]) + b_ref[...]
    i = jax.nn.sigmoid(g[:, 0:hdim])
    f = jax.nn.sigmoid(g[:, hdim:2 * hdim])
    gg = jnp.tanh(g[:, 2 * hdim:3 * hdim])
    o = jax.nn.sigmoid(g[:, 3 * hdim:4 * hdim])
    cn = f * c_ref[...] + i * gg
    hout_ref[...] = o * jnp.tanh(cn)
    cout_ref[...] = cn


def _lstm(p, H, C, fb, ff):
    B, m, c = fb.shape
    hdim = H.shape[2]
    fn = pl.pallas_call(
        functools.partial(_lstm_kernel, hdim=hdim),
        out_shape=(jax.ShapeDtypeStruct((m, hdim), _F32),
                   jax.ShapeDtypeStruct((m, hdim), _F32)),
    )
    return jax.vmap(fn, in_axes=(0, 0, 0, 0, None, None, None))(
        fb, ff, H, C, p['Wx'], p['Wh'], p['b'].reshape(1, -1))


def _interp(xc, posf, poscT, d_ref, m, n, k):
    D = ((posf[:, 0:1] - poscT[0:1, :]) ** 2
         + (posf[:, 1:2] - poscT[1:2, :]) ** 2)
    d_ref[...] = D + (posf[:, 2:3] - poscT[2:3, :]) ** 2
    lane_mn = jax.lax.broadcasted_iota(jnp.int32, (m, n), 1)
    c = xc.shape[1]

    def body(_, carry):
        acc, wsum = carry
        D = d_ref[...]
        oh, mn, idx = _first_min_onehot(D, lane_mn, n)
        d_ref[...] = jnp.where(lane_mn == idx, _BIG, D)
        w = 1.0 / (mn + 1e-2)
        g = _dot(oh, xc)
        return acc + w * g, wsum + w

    acc, wsum = jax.lax.fori_loop(
        0, k, body, (jnp.zeros((m, c), _F32), jnp.zeros((m, 1), _F32)))
    return acc / wsum


def _fp_kernel(xc_ref, posf_ref, poscT_ref, xskip_ref,
               w1_ref, b1_ref, w2_ref, b2_ref, out_ref, d_ref, *, m, n, k):
    interp = _interp(xc_ref[...], posf_ref[...], poscT_ref[...], d_ref, m, n, k)
    h = jnp.concatenate([interp, xskip_ref[...]], axis=1)
    h = jnp.maximum(_dot(h, w1_ref[...]) + b1_ref[...], 0.0)
    h = jnp.maximum(_dot(h, w2_ref[...]) + b2_ref[...], 0.0)
    out_ref[...] = h


def _fp(layers, x_c, pos_c, x_skip, pos_f, k):
    B, m, _ = pos_f.shape
    n = pos_c.shape[1]
    (w1, b1), (w2, b2) = layers
    poscT = jnp.transpose(pos_c, (0, 2, 1))
    fn = pl.pallas_call(
        functools.partial(_fp_kernel, m=m, n=n, k=k),
        out_shape=jax.ShapeDtypeStruct((m, w2.shape[1]), _F32),
        scratch_shapes=[pltpu.VMEM((m, n), _F32)],
    )
    return jax.vmap(fn, in_axes=(0, 0, 0, 0, None, None, None, None))(
        x_c, pos_f, poscT, x_skip, w1, b1.reshape(1, -1), w2, b2.reshape(1, -1))


def _fpns_cls_kernel(xc_ref, posf_ref, poscT_ref,
                     w1_ref, b1_ref, w2_ref, b2_ref,
                     c1_ref, c2_ref, c3_ref, c4_ref, out_ref, d_ref,
                     *, m, n, k):
    interp = _interp(xc_ref[...], posf_ref[...], poscT_ref[...], d_ref, m, n, k)
    h = jnp.maximum(_dot(interp, w1_ref[...]) + b1_ref[...], 0.0)
    h = jnp.maximum(_dot(h, w2_ref[...]) + b2_ref[...], 0.0)
    h = _dot(h, c1_ref[...])
    h = _dot(h, c2_ref[...])
    h = _dot(h, c3_ref[...])
    h = _dot(h, c4_ref[...])
    out_ref[...] = posf_ref[...] + h


def _fpns_cls(layers, cls, x_c, pos_c, pos_f, k):
    B, m, _ = pos_f.shape
    n = pos_c.shape[1]
    (w1, b1), (w2, b2) = layers
    c1, c2, c3, c4 = cls
    poscT = jnp.transpose(pos_c, (0, 2, 1))
    fn = pl.pallas_call(
        functools.partial(_fpns_cls_kernel, m=m, n=n, k=k),
        out_shape=jax.ShapeDtypeStruct((m, 3), _F32),
        scratch_shapes=[pltpu.VMEM((m, n), _F32)],
    )
    return jax.vmap(fn, in_axes=(0, 0, 0) + (None,) * 8)(
        x_c, pos_f, poscT, w1, b1.reshape(1, -1), w2, b2.reshape(1, -1),
        c1, c2, c3, c4)


def kernel(input_xyz, num_pred, params):
    p = params
    T, B, _, N = input_xyz.shape
    frames = jnp.transpose(input_xyz, (0, 1, 3, 2))
    N1, N2, N3 = N // 16, N // 32, N // 64

    def encode(fr):
        f1, x1 = _sa(p['sa1'], fr, fr, N1, 32)
        f2, x2 = _sa(p['sa2'], f1, x1, N2, 16)
        f3, x3 = _sa(p['sa3'], f2, x2, N3, 8)
        return (f1, x1, f2, x2, f3, x3)

    encs = [encode(frames[t]) for t in range(T)]

    st = (jnp.zeros((B, N1, 128), _F32), jnp.zeros((B, N1, 128), _F32),
          jnp.zeros((B, N2, 256), _F32), jnp.zeros((B, N2, 256), _F32),
          jnp.zeros((B, N3, 512), _F32), jnp.zeros((B, N3, 512), _F32))

    def tstep(st, prev, cur, nxt):
        H1, C1, H2, C2, H3, C3 = st
        fb1 = _lpt(p['gat1'], cur[0], prev[0], cur[1], prev[1], 16)
        ff1 = _lpt(p['gat1'], cur[0], nxt[0], cur[1], nxt[1], 16)
        fb2 = _lpt(p['gat2'], cur[2], prev[2], cur[3], prev[3], 16)
        ff2 = _lpt(p['gat2'], cur[2], nxt[2], cur[3], nxt[3], 16)
        fb3 = _lpt(p['gat3'], cur[4], prev[4], cur[5], prev[5], 8)
        ff3 = _lpt(p['gat3'], cur[4], nxt[4], cur[5], nxt[5], 8)
        H1, C1 = _lstm(p['lstm1'], H1, C1, fb1, ff1)
        H2, C2 = _lstm(p['lstm2'], H2, C2, fb2, ff2)
        H3, C3 = _lstm(p['lstm3'], H3, C3, fb3, ff3)
        return (H1, C1, H2, C2, H3, C3)

    st = tstep(st, encs[0], encs[0], encs[1])
    for i in range(1, T):
        nxt = encs[i + 1] if i < T - 1 else encs[i]
        st = tstep(st, encs[i - 1], encs[i], nxt)

    def decode(st, e, fine_xyz):
        H1, _, H2, _, H3, _ = st
        x2 = _fp(p['fp32'], H3, e[5], H2, e[3], 8)
        x1 = _fp(p['fp21'], x2, e[3], H1, e[1], 16)
        return _fpns_cls(p['fp10'], p['cls'], x1, e[1], fine_xyz, 32)

    num_steps = 2
    pc_next = decode(st, encs[-1], frames[-1])
    preds = [pc_next]
    for _ in range(1, num_steps):
        e_new = encode(pc_next)
        st = tstep(st, encs[-1], e_new, e_new)
        encs.append(e_new)
        pc_next = decode(st, e_new, pc_next)
        preds.append(pc_next)
    return jnp.stack(preds)


# speedup vs baseline: 5.7424x; 1.5267x over previous
import functools

import jax
import jax.numpy as jnp
import numpy as np
from jax.experimental import pallas as pl
from jax.experimental.pallas import tpu as pltpu

_F32 = jnp.float32
_BIG = 3.0e38
_PREC = jax.lax.Precision.DEFAULT


def _dot(a, b):
    return jax.lax.dot_general(a, b, (((1,), (0,)), ((), ())),
                               preferred_element_type=_F32, precision=_PREC)


def _first_min_onehot(D, lane_iota, n):
    mn = jnp.min(D, axis=1, keepdims=True)
    idx = jnp.min(jnp.where(D == mn, lane_iota, n), axis=1, keepdims=True)
    oh = (lane_iota == idx).astype(_F32)
    return oh, mn, idx


def _sa_kernel(featxyz_ref, xyz_ref, xyzT_ref,
               w1_ref, b1_ref, w2_ref, b2_ref, w3_ref, b3_ref,
               fout_ref, cen_ref, d_ref,
               *, n, m, k, cf):
    xyzT = xyzT_ref[...]
    lane_n = jax.lax.broadcasted_iota(jnp.int32, (1, n), 1)

    d0 = jnp.sum((xyzT - xyzT[:, 0:1]) ** 2, axis=0, keepdims=True)
    d_ref[0:1, :] = d0
    cen_ref[0:1, :] = xyz_ref[0:1, :]

    def fps_body(i, dists):
        mx = jnp.max(dists)
        idx = jnp.min(jnp.where(dists == mx, lane_n, n))
        cen_ref[pl.ds(i, 1), :] = xyz_ref[pl.ds(idx, 1), :]
        mask = (lane_n == idx).astype(_F32)
        col = jnp.sum(xyzT * mask, axis=1, keepdims=True)
        dnew = jnp.sum((xyzT - col) ** 2, axis=0, keepdims=True)
        d_ref[pl.ds(i, 1), :] = dnew
        return jnp.minimum(dists, dnew)

    jax.lax.fori_loop(1, m, fps_body, d0)

    lane_mn = jax.lax.broadcasted_iota(jnp.int32, (m, n), 1)
    cen = cen_ref[...]
    cenpad = jnp.concatenate([jnp.zeros((m, cf), _F32), cen], axis=1)
    featxyz = featxyz_ref[...]
    w1, b1 = w1_ref[...], b1_ref[...]
    w2, b2 = w2_ref[...], b2_ref[...]
    w3, b3 = w3_ref[...], b3_ref[...]

    def nbr_body(_, acc):
        D = d_ref[...]
        oh, _, idx = _first_min_onehot(D, lane_mn, n)
        d_ref[...] = jnp.where(lane_mn == idx, _BIG, D)
        g = _dot(oh, featxyz) - cenpad
        a = jnp.maximum(_dot(g, w1) + b1, 0.0)
        a = jnp.maximum(_dot(a, w2) + b2, 0.0)
        a = jnp.maximum(_dot(a, w3) + b3, 0.0)
        return jnp.maximum(acc, a)

    cout = w3.shape[1]
    fout_ref[...] = jax.lax.fori_loop(
        0, k, nbr_body, jnp.zeros((m, cout), _F32))


def _sa(layers, feat, xyz, m, k):
    B, n, cf = feat.shape
    (w1, b1), (w2, b2), (w3, b3) = layers
    cout = w3.shape[1]
    featxyz = jnp.concatenate([feat, xyz], axis=2)
    xyzT = jnp.transpose(xyz, (0, 2, 1))
    fn = pl.pallas_call(
        functools.partial(_sa_kernel, n=n, m=m, k=k, cf=cf),
        out_shape=(jax.ShapeDtypeStruct((m, cout), _F32),
                   jax.ShapeDtypeStruct((m, 3), _F32)),
        scratch_shapes=[pltpu.VMEM((m, n), _F32)],
    )
    fout, cen = jax.vmap(fn, in_axes=(0, 0, 0) + (None,) * 6)(
        featxyz, xyz, xyzT,
        w1, b1.reshape(1, -1), w2, b2.reshape(1, -1), w3, b3.reshape(1, -1))
    return fout, cen


def _lpt_kernel(fcur_ref, fsrc_ref, qxyz_ref, sxyzT_ref,
                wq_ref, wk_ref, wv_ref, out_ref, d_ref, s_ref, i_ref,
                *, m, n, k, c):
    qxyz = qxyz_ref[...]
    sxyzT = sxyzT_ref[...]
    D = ((qxyz[:, 0:1] - sxyzT[0:1, :]) ** 2
         + (qxyz[:, 1:2] - sxyzT[1:2, :]) ** 2)
    d_ref[...] = D + (qxyz[:, 2:3] - sxyzT[2:3, :]) ** 2
    lane_mn = jax.lax.broadcasted_iota(jnp.int32, (m, n), 1)

    q = _dot(fcur_ref[...], wq_ref[...])
    K = _dot(fsrc_ref[...], wk_ref[...])
    V = _dot(fsrc_ref[...], wv_ref[...])
    S = jax.lax.dot_general(q, K, (((1,), (1,)), ((), ())),
                            preferred_element_type=_F32,
                            precision=_PREC)

    def sel_body(j, _):
        D = d_ref[...]
        oh, _, idx = _first_min_onehot(D, lane_mn, n)
        d_ref[...] = jnp.where(lane_mn == idx, _BIG, D)
        s_ref[j] = jnp.sum(S * oh, axis=1, keepdims=True)
        i_ref[j] = idx
        return 0

    jax.lax.fori_loop(0, k, sel_body, 0)

    s = s_ref[...] / np.sqrt(c)
    e = jnp.exp(s - jnp.max(s, axis=0, keepdims=True))
    s_ref[...] = e / jnp.sum(e, axis=0, keepdims=True)

    def mix_body(j, A):
        return A + s_ref[j] * (lane_mn == i_ref[j]).astype(_F32)

    A = jax.lax.fori_loop(0, k, mix_body, jnp.zeros((m, n), _F32))
    out_ref[...] = _dot(A, V)


def _lpt(p, f_cur, f_src, xyz_cur, xyz_src, k):
    B, m, c = f_cur.shape
    n = f_src.shape[1]
    sxyzT = jnp.transpose(xyz_src, (0, 2, 1))
    fn = pl.pallas_call(
        functools.partial(_lpt_kernel, m=m, n=n, k=k, c=c),
        out_shape=jax.ShapeDtypeStruct((m, c), _F32),
        scratch_shapes=[pltpu.VMEM((m, n), _F32),
                        pltpu.VMEM((k, m, 1), _F32),
                        pltpu.VMEM((k, m, 1), jnp.int32)],
    )
    return jax.vmap(fn, in_axes=(0, 0, 0, 0, None, None, None))(
        f_cur, f_src, xyz_cur, sxyzT, p['Wq'], p['Wk'], p['Wv'])


def _lstm_kernel(fb_ref, ff_ref, h_ref, c_ref, wx_ref, wh_ref, b_ref,
                 hout_ref, cout_ref, *, hdim):
    x = jnp.concatenate([fb_ref[...], ff_ref[...]], axis=1)
    g = _dot(x, wx_ref[...]) + _dot(h_ref[...], wh_ref[...]) + b_ref[...]
    i = jax.nn.sigmoid(g[:, 0:hdim])
    f = jax.nn.sigmoid(g[:, hdim:2 * hdim])
    gg = jnp.tanh(g[:, 2 * hdim:3 * hdim])
    o = jax.nn.sigmoid(g[:, 3 * hdim:4 * hdim])
    cn = f * c_ref[...] + i * gg
    hout_ref[...] = o * jnp.tanh(cn)
    cout_ref[...] = cn


def _lstm(p, H, C, fb, ff):
    B, m, c = fb.shape
    hdim = H.shape[2]
    fn = pl.pallas_call(
        functools.partial(_lstm_kernel, hdim=hdim),
        out_shape=(jax.ShapeDtypeStruct((m, hdim), _F32),
                   jax.ShapeDtypeStruct((m, hdim), _F32)),
    )
    return jax.vmap(fn, in_axes=(0, 0, 0, 0, None, None, None))(
        fb, ff, H, C, p['Wx'], p['Wh'], p['b'].reshape(1, -1))


def _interp(xc, posf, poscT, d_ref, m, n, k):
    D = ((posf[:, 0:1] - poscT[0:1, :]) ** 2
         + (posf[:, 1:2] - poscT[1:2, :]) ** 2)
    d_ref[...] = D + (posf[:, 2:3] - poscT[2:3, :]) ** 2
    lane_mn = jax.lax.broadcasted_iota(jnp.int32, (m, n), 1)

    def body(_, carry):
        W, wsum = carry
        D = d_ref[...]
        oh, mn, idx = _first_min_onehot(D, lane_mn, n)
        d_ref[...] = jnp.where(lane_mn == idx, _BIG, D)
        w = 1.0 / (mn + 1e-2)
        return W + w * oh, wsum + w

    W, wsum = jax.lax.fori_loop(
        0, k, body, (jnp.zeros((m, n), _F32), jnp.zeros((m, 1), _F32)))
    return _dot(W, xc) / wsum


def _fp_kernel(xc_ref, posf_ref, poscT_ref, xskip_ref,
               w1_ref, b1_ref, w2_ref, b2_ref, out_ref, d_ref, *, m, n, k):
    interp = _interp(xc_ref[...], posf_ref[...], poscT_ref[...], d_ref, m, n, k)
    h = jnp.concatenate([interp, xskip_ref[...]], axis=1)
    h = jnp.maximum(_dot(h, w1_ref[...]) + b1_ref[...], 0.0)
    h = jnp.maximum(_dot(h, w2_ref[...]) + b2_ref[...], 0.0)
    out_ref[...] = h


def _fp(layers, x_c, pos_c, x_skip, pos_f, k):
    B, m, _ = pos_f.shape
    n = pos_c.shape[1]
    (w1, b1), (w2, b2) = layers
    poscT = jnp.transpose(pos_c, (0, 2, 1))
    fn = pl.pallas_call(
        functools.partial(_fp_kernel, m=m, n=n, k=k),
        out_shape=jax.ShapeDtypeStruct((m, w2.shape[1]), _F32),
        scratch_shapes=[pltpu.VMEM((m, n), _F32)],
    )
    return jax.vmap(fn, in_axes=(0, 0, 0, 0, None, None, None, None))(
        x_c, pos_f, poscT, x_skip, w1, b1.reshape(1, -1), w2, b2.reshape(1, -1))


def _fpns_cls_kernel(xc_ref, posf_ref, poscT_ref,
                     w1_ref, b1_ref, w2_ref, b2_ref,
                     c1_ref, c2_ref, c3_ref, c4_ref, out_ref, d_ref,
                     *, m, n, k):
    interp = _interp(xc_ref[...], posf_ref[...], poscT_ref[...], d_ref, m, n, k)
    h = jnp.maximum(_dot(interp, w1_ref[...]) + b1_ref[...], 0.0)
    h = jnp.maximum(_dot(h, w2_ref[...]) + b2_ref[...], 0.0)
    h = _dot(h, c1_ref[...])
    h = _dot(h, c2_ref[...])
    h = _dot(h, c3_ref[...])
    h = _dot(h, c4_ref[...])
    out_ref[...] = posf_ref[...] + h


def _fpns_cls(layers, cls, x_c, pos_c, pos_f, k):
    B, m, _ = pos_f.shape
    n = pos_c.shape[1]
    (w1, b1), (w2, b2) = layers
    c1, c2, c3, c4 = cls
    poscT = jnp.transpose(pos_c, (0, 2, 1))
    fn = pl.pallas_call(
        functools.partial(_fpns_cls_kernel, m=m, n=n, k=k),
        out_shape=jax.ShapeDtypeStruct((m, 3), _F32),
        scratch_shapes=[pltpu.VMEM((m, n), _F32)],
    )
    return jax.vmap(fn, in_axes=(0, 0, 0) + (None,) * 8)(
        x_c, pos_f, poscT, w1, b1.reshape(1, -1), w2, b2.reshape(1, -1),
        c1, c2, c3, c4)


def kernel(input_xyz, num_pred, params):
    p = params
    T, B, _, N = input_xyz.shape
    frames = jnp.transpose(input_xyz, (0, 1, 3, 2))
    N1, N2, N3 = N // 16, N // 32, N // 64

    def encode(fr):
        f1, x1 = _sa(p['sa1'], fr, fr, N1, 32)
        f2, x2 = _sa(p['sa2'], f1, x1, N2, 16)
        f3, x3 = _sa(p['sa3'], f2, x2, N3, 8)
        return (f1, x1, f2, x2, f3, x3)

    encs = [encode(frames[t]) for t in range(T)]

    st = (jnp.zeros((B, N1, 128), _F32), jnp.zeros((B, N1, 128), _F32),
          jnp.zeros((B, N2, 256), _F32), jnp.zeros((B, N2, 256), _F32),
          jnp.zeros((B, N3, 512), _F32), jnp.zeros((B, N3, 512), _F32))

    def tstep(st, prev, cur, nxt):
        H1, C1, H2, C2, H3, C3 = st
        fb1 = _lpt(p['gat1'], cur[0], prev[0], cur[1], prev[1], 16)
        ff1 = _lpt(p['gat1'], cur[0], nxt[0], cur[1], nxt[1], 16)
        fb2 = _lpt(p['gat2'], cur[2], prev[2], cur[3], prev[3], 16)
        ff2 = _lpt(p['gat2'], cur[2], nxt[2], cur[3], nxt[3], 16)
        fb3 = _lpt(p['gat3'], cur[4], prev[4], cur[5], prev[5], 8)
        ff3 = _lpt(p['gat3'], cur[4], nxt[4], cur[5], nxt[5], 8)
        H1, C1 = _lstm(p['lstm1'], H1, C1, fb1, ff1)
        H2, C2 = _lstm(p['lstm2'], H2, C2, fb2, ff2)
        H3, C3 = _lstm(p['lstm3'], H3, C3, fb3, ff3)
        return (H1, C1, H2, C2, H3, C3)

    st = tstep(st, encs[0], encs[0], encs[1])
    for i in range(1, T):
        nxt = encs[i + 1] if i < T - 1 else encs[i]
        st = tstep(st, encs[i - 1], encs[i], nxt)

    def decode(st, e, fine_xyz):
        H1, _, H2, _, H3, _ = st
        x2 = _fp(p['fp32'], H3, e[5], H2, e[3], 8)
        x1 = _fp(p['fp21'], x2, e[3], H1, e[1], 16)
        return _fpns_cls(p['fp10'], p['cls'], x1, e[1], fine_xyz, 32)

    num_steps = 2
    pc_next = decode(st, encs[-1], frames[-1])
    preds = [pc_next]
    for _ in range(1, num_steps):
        e_new = encode(pc_next)
        st = tstep(st, encs[-1], e_new, e_new)
        encs.append(e_new)
        pc_next = decode(st, e_new, pc_next)
        preds.append(pc_next)
    return jnp.stack(preds)


# speedup vs baseline: 8.9421x; 1.5572x over previous
import functools

import jax
import jax.numpy as jnp
import numpy as np
from jax.experimental import pallas as pl
from jax.experimental.pallas import tpu as pltpu

_F32 = jnp.float32
_BIG = 3.0e38
_PREC = jax.lax.Precision.DEFAULT


def _dot(a, b):
    return jax.lax.dot_general(a, b, (((1,), (0,)), ((), ())),
                               preferred_element_type=_F32, precision=_PREC)


def _first_min_onehot(D, lane_iota, n):
    mn = jnp.min(D, axis=1, keepdims=True)
    idx = jnp.min(jnp.where(D == mn, lane_iota, n), axis=1, keepdims=True)
    oh = (lane_iota == idx).astype(_F32)
    return oh, mn, idx


def _fps_kernel(xyzT_ref, cen_ref, *, n, m):
    xT = xyzT_ref[...]
    xs, ys, zs = xT[:, 0:1, :], xT[:, 1:2, :], xT[:, 2:3, :]
    x0, y0, z0 = xs[:, :, 0:1], ys[:, :, 0:1], zs[:, :, 0:1]
    d0 = (xs - x0) ** 2 + (ys - y0) ** 2 + (zs - z0) ** 2
    cen_ref[:, 0:1, :] = jnp.concatenate([x0, y0, z0], axis=2)
    iota = jax.lax.broadcasted_iota(jnp.int32, (1, 1, n), 2)

    def body(i, dists):
        mx = jnp.max(dists, axis=2, keepdims=True)
        sel = jnp.min(jnp.where(dists == mx, iota, n), axis=2, keepdims=True)
        mask = (iota == sel).astype(_F32)
        xc = jnp.sum(xs * mask, axis=2, keepdims=True)
        yc = jnp.sum(ys * mask, axis=2, keepdims=True)
        zc = jnp.sum(zs * mask, axis=2, keepdims=True)
        dnew = (xs - xc) ** 2 + (ys - yc) ** 2 + (zs - zc) ** 2
        cen_ref[:, pl.ds(i, 1), :] = jnp.concatenate([xc, yc, zc], axis=2)
        return jnp.minimum(dists, dnew)

    jax.lax.fori_loop(1, m, body, d0)


def _sa_kernel(featxyz_ref, cen_ref, xyzT_ref,
               w1_ref, b1_ref, w2_ref, b2_ref, w3_ref, b3_ref,
               fout_ref, d_ref,
               *, n, m, k, cf):
    cen = cen_ref[...]
    sxyzT = xyzT_ref[...]
    D = ((cen[:, 0:1] - sxyzT[0:1, :]) ** 2
         + (cen[:, 1:2] - sxyzT[1:2, :]) ** 2)
    d_ref[...] = D + (cen[:, 2:3] - sxyzT[2:3, :]) ** 2

    lane_mn = jax.lax.broadcasted_iota(jnp.int32, (m, n), 1)
    cenpad = jnp.concatenate([jnp.zeros((m, cf), _F32), cen], axis=1)
    featxyz = featxyz_ref[...]
    w1, b1 = w1_ref[...], b1_ref[...]
    w2, b2 = w2_ref[...], b2_ref[...]
    w3, b3 = w3_ref[...], b3_ref[...]

    def nbr_body(_, acc):
        D = d_ref[...]
        oh, _, idx = _first_min_onehot(D, lane_mn, n)
        d_ref[...] = jnp.where(lane_mn == idx, _BIG, D)
        g = _dot(oh, featxyz) - cenpad
        a = jnp.maximum(_dot(g, w1) + b1, 0.0)
        a = jnp.maximum(_dot(a, w2) + b2, 0.0)
        a = jnp.maximum(_dot(a, w3) + b3, 0.0)
        return jnp.maximum(acc, a)

    cout = w3.shape[1]
    fout_ref[...] = jax.lax.fori_loop(
        0, k, nbr_body, jnp.zeros((m, cout), _F32))


def _sa(layers, feat, xyz, m, k):
    G, n, cf = feat.shape
    (w1, b1), (w2, b2), (w3, b3) = layers
    cout = w3.shape[1]
    featxyz = jnp.concatenate([feat, xyz], axis=2)
    xyzT = jnp.transpose(xyz, (0, 2, 1))
    cen = pl.pallas_call(
        functools.partial(_fps_kernel, n=n, m=m),
        out_shape=jax.ShapeDtypeStruct((G, m, 3), _F32),
    )(xyzT)
    fn = pl.pallas_call(
        functools.partial(_sa_kernel, n=n, m=m, k=k, cf=cf),
        out_shape=jax.ShapeDtypeStruct((m, cout), _F32),
        scratch_shapes=[pltpu.VMEM((m, n), _F32)],
    )
    fout = jax.vmap(fn, in_axes=(0, 0, 0) + (None,) * 6)(
        featxyz, cen, xyzT,
        w1, b1.reshape(1, -1), w2, b2.reshape(1, -1), w3, b3.reshape(1, -1))
    return fout, cen


def _lpt_kernel(fcur_ref, fsrc_ref, qxyz_ref, sxyzT_ref,
                wq_ref, wk_ref, wv_ref, out_ref, d_ref, s_ref, i_ref,
                *, m, n, k, c):
    qxyz = qxyz_ref[...]
    sxyzT = sxyzT_ref[...]
    D = ((qxyz[:, 0:1] - sxyzT[0:1, :]) ** 2
         + (qxyz[:, 1:2] - sxyzT[1:2, :]) ** 2)
    d_ref[...] = D + (qxyz[:, 2:3] - sxyzT[2:3, :]) ** 2
    lane_mn = jax.lax.broadcasted_iota(jnp.int32, (m, n), 1)

    q = _dot(fcur_ref[...], wq_ref[...])
    K = _dot(fsrc_ref[...], wk_ref[...])
    V = _dot(fsrc_ref[...], wv_ref[...])
    S = jax.lax.dot_general(q, K, (((1,), (1,)), ((), ())),
                            preferred_element_type=_F32,
                            precision=_PREC)

    def sel_body(j, _):
        D = d_ref[...]
        oh, _, idx = _first_min_onehot(D, lane_mn, n)
        d_ref[...] = jnp.where(lane_mn == idx, _BIG, D)
        s_ref[j] = jnp.sum(S * oh, axis=1, keepdims=True)
        i_ref[j] = idx
        return 0

    jax.lax.fori_loop(0, k, sel_body, 0)

    s = s_ref[...] / np.sqrt(c)
    e = jnp.exp(s - jnp.max(s, axis=0, keepdims=True))
    s_ref[...] = e / jnp.sum(e, axis=0, keepdims=True)

    def mix_body(j, A):
        return A + s_ref[j] * (lane_mn == i_ref[j]).astype(_F32)

    A = jax.lax.fori_loop(0, k, mix_body, jnp.zeros((m, n), _F32))
    out_ref[...] = _dot(A, V)


def _lpt(p, f_cur, f_src, xyz_cur, xyz_src, k):
    B, m, c = f_cur.shape
    n = f_src.shape[1]
    sxyzT = jnp.transpose(xyz_src, (0, 2, 1))
    fn = pl.pallas_call(
        functools.partial(_lpt_kernel, m=m, n=n, k=k, c=c),
        out_shape=jax.ShapeDtypeStruct((m, c), _F32),
        scratch_shapes=[pltpu.VMEM((m, n), _F32),
                        pltpu.VMEM((k, m, 1), _F32),
                        pltpu.VMEM((k, m, 1), jnp.int32)],
    )
    return jax.vmap(fn, in_axes=(0, 0, 0, 0, None, None, None))(
        f_cur, f_src, xyz_cur, sxyzT, p['Wq'], p['Wk'], p['Wv'])


def _lstm_kernel(fb_ref, ff_ref, h_ref, c_ref, wx_ref, wh_ref, b_ref,
                 hout_ref, cout_ref, *, hdim):
    x = jnp.concatenate([fb_ref[...], ff_ref[...]], axis=1)
    g = _dot(x, wx_ref[...]) + _dot(h_ref[...], wh_ref[...]) + b_ref[...]
    i = jax.nn.sigmoid(g[:, 0:hdim])
    f = jax.nn.sigmoid(g[:, hdim:2 * hdim])
    gg = jnp.tanh(g[:, 2 * hdim:3 * hdim])
    o = jax.nn.sigmoid(g[:, 3 * hdim:4 * hdim])
    cn = f * c_ref[...] + i * gg
    hout_ref[...] = o * jnp.tanh(cn)
    cout_ref[...] = cn


def _lstm(p, H, C, fb, ff):
    B, m, c = fb.shape
    hdim = H.shape[2]
    fn = pl.pallas_call(
        functools.partial(_lstm_kernel, hdim=hdim),
        out_shape=(jax.ShapeDtypeStruct((m, hdim), _F32),
                   jax.ShapeDtypeStruct((m, hdim), _F32)),
    )
    return jax.vmap(fn, in_axes=(0, 0, 0, 0, None, None, None))(
        fb, ff, H, C, p['Wx'], p['Wh'], p['b'].reshape(1, -1))


def _interp(xc, posf, poscT, d_ref, m, n, k):
    D = ((posf[:, 0:1] - poscT[0:1, :]) ** 2
         + (posf[:, 1:2] - poscT[1:2, :]) ** 2)
    d_ref[...] = D + (posf[:, 2:3] - poscT[2:3, :]) ** 2
    lane_mn = jax.lax.broadcasted_iota(jnp.int32, (m, n), 1)

    def body(_, carry):
        W, wsum = carry
        D = d_ref[...]
        oh, mn, idx = _first_min_onehot(D, lane_mn, n)
        d_ref[...] = jnp.where(lane_mn == idx, _BIG, D)
        w = 1.0 / (mn + 1e-2)
        return W + w * oh, wsum + w

    W, wsum = jax.lax.fori_loop(
        0, k, body, (jnp.zeros((m, n), _F32), jnp.zeros((m, 1), _F32)))
    return _dot(W, xc) / wsum


def _fp_kernel(xc_ref, posf_ref, poscT_ref, xskip_ref,
               w1_ref, b1_ref, w2_ref, b2_ref, out_ref, d_ref, *, m, n, k):
    interp = _interp(xc_ref[...], posf_ref[...], poscT_ref[...], d_ref, m, n, k)
    h = jnp.concatenate([interp, xskip_ref[...]], axis=1)
    h = jnp.maximum(_dot(h, w1_ref[...]) + b1_ref[...], 0.0)
    h = jnp.maximum(_dot(h, w2_ref[...]) + b2_ref[...], 0.0)
    out_ref[...] = h


def _fp(layers, x_c, pos_c, x_skip, pos_f, k):
    B, m, _ = pos_f.shape
    n = pos_c.shape[1]
    (w1, b1), (w2, b2) = layers
    poscT = jnp.transpose(pos_c, (0, 2, 1))
    fn = pl.pallas_call(
        functools.partial(_fp_kernel, m=m, n=n, k=k),
        out_shape=jax.ShapeDtypeStruct((m, w2.shape[1]), _F32),
        scratch_shapes=[pltpu.VMEM((m, n), _F32)],
    )
    return jax.vmap(fn, in_axes=(0, 0, 0, 0, None, None, None, None))(
        x_c, pos_f, poscT, x_skip, w1, b1.reshape(1, -1), w2, b2.reshape(1, -1))


def _fpns_cls_kernel(xc_ref, posf_ref, poscT_ref,
                     w1_ref, b1_ref, w2_ref, b2_ref,
                     c1_ref, c2_ref, c3_ref, c4_ref, out_ref, d_ref,
                     *, m, n, k):
    interp = _interp(xc_ref[...], posf_ref[...], poscT_ref[...], d_ref, m, n, k)
    h = jnp.maximum(_dot(interp, w1_ref[...]) + b1_ref[...], 0.0)
    h = jnp.maximum(_dot(h, w2_ref[...]) + b2_ref[...], 0.0)
    h = _dot(h, c1_ref[...])
    h = _dot(h, c2_ref[...])
    h = _dot(h, c3_ref[...])
    h = _dot(h, c4_ref[...])
    out_ref[...] = posf_ref[...] + h


def _fpns_cls(layers, cls, x_c, pos_c, pos_f, k):
    B, m, _ = pos_f.shape
    n = pos_c.shape[1]
    (w1, b1), (w2, b2) = layers
    c1, c2, c3, c4 = cls
    poscT = jnp.transpose(pos_c, (0, 2, 1))
    fn = pl.pallas_call(
        functools.partial(_fpns_cls_kernel, m=m, n=n, k=k),
        out_shape=jax.ShapeDtypeStruct((m, 3), _F32),
        scratch_shapes=[pltpu.VMEM((m, n), _F32)],
    )
    return jax.vmap(fn, in_axes=(0, 0, 0) + (None,) * 8)(
        x_c, pos_f, poscT, w1, b1.reshape(1, -1), w2, b2.reshape(1, -1),
        c1, c2, c3, c4)


def kernel(input_xyz, num_pred, params):
    p = params
    T, B, _, N = input_xyz.shape
    frames = jnp.transpose(input_xyz, (0, 1, 3, 2))
    N1, N2, N3 = N // 16, N // 32, N // 64

    def encode(fr):
        f1, x1 = _sa(p['sa1'], fr, fr, N1, 32)
        f2, x2 = _sa(p['sa2'], f1, x1, N2, 16)
        f3, x3 = _sa(p['sa3'], f2, x2, N3, 8)
        return (f1, x1, f2, x2, f3, x3)

    e_all = encode(frames.reshape(T * B, N, 3))
    encs = [tuple(a.reshape((T, B) + a.shape[1:])[t] for a in e_all)
            for t in range(T)]

    st = (jnp.zeros((B, N1, 128), _F32), jnp.zeros((B, N1, 128), _F32),
          jnp.zeros((B, N2, 256), _F32), jnp.zeros((B, N2, 256), _F32),
          jnp.zeros((B, N3, 512), _F32), jnp.zeros((B, N3, 512), _F32))

    def tstep(st, prev, cur, nxt):
        H1, C1, H2, C2, H3, C3 = st
        fb1 = _lpt(p['gat1'], cur[0], prev[0], cur[1], prev[1], 16)
        ff1 = _lpt(p['gat1'], cur[0], nxt[0], cur[1], nxt[1], 16)
        fb2 = _lpt(p['gat2'], cur[2], prev[2], cur[3], prev[3], 16)
        ff2 = _lpt(p['gat2'], cur[2], nxt[2], cur[3], nxt[3], 16)
        fb3 = _lpt(p['gat3'], cur[4], prev[4], cur[5], prev[5], 8)
        ff3 = _lpt(p['gat3'], cur[4], nxt[4], cur[5], nxt[5], 8)
        H1, C1 = _lstm(p['lstm1'], H1, C1, fb1, ff1)
        H2, C2 = _lstm(p['lstm2'], H2, C2, fb2, ff2)
        H3, C3 = _lstm(p['lstm3'], H3, C3, fb3, ff3)
        return (H1, C1, H2, C2, H3, C3)

    st = tstep(st, encs[0], encs[0], encs[1])
    for i in range(1, T):
        nxt = encs[i + 1] if i < T - 1 else encs[i]
        st = tstep(st, encs[i - 1], encs[i], nxt)

    def decode(st, e, fine_xyz):
        H1, _, H2, _, H3, _ = st
        x2 = _fp(p['fp32'], H3, e[5], H2, e[3], 8)
        x1 = _fp(p['fp21'], x2, e[3], H1, e[1], 16)
        return _fpns_cls(p['fp10'], p['cls'], x1, e[1], fine_xyz, 32)

    num_steps = 2
    pc_next = decode(st, encs[-1], frames[-1])
    preds = [pc_next]
    for _ in range(1, num_steps):
        e_new = encode(pc_next)
        st = tstep(st, encs[-1], e_new, e_new)
        encs.append(e_new)
        pc_next = decode(st, e_new, pc_next)
        preds.append(pc_next)
    return jnp.stack(preds)


# speedup vs baseline: 9.9436x; 1.1120x over previous
import functools

import jax
import jax.numpy as jnp
import numpy as np
from jax.experimental import pallas as pl
from jax.experimental.pallas import tpu as pltpu

_F32 = jnp.float32
_BIG = 3.0e38
_PREC = jax.lax.Precision.DEFAULT


def _dot(a, b):
    return jax.lax.dot_general(a, b, (((1,), (0,)), ((), ())),
                               preferred_element_type=_F32, precision=_PREC)


def _first_min_onehot(D, lane_iota, n):
    mn = jnp.min(D, axis=1, keepdims=True)
    idx = jnp.min(jnp.where(D == mn, lane_iota, n), axis=1, keepdims=True)
    oh = (lane_iota == idx).astype(_F32)
    return oh, mn, idx


def _fps_kernel(xs_ref, ys_ref, zs_ref, cx_ref, cy_ref, cz_ref, *, n, m):
    xs, ys, zs = xs_ref[...], ys_ref[...], zs_ref[...]
    x0, y0, z0 = xs[:, 0:1], ys[:, 0:1], zs[:, 0:1]
    iota = jax.lax.broadcasted_iota(jnp.int32, (1, n), 1)
    lane_m = jax.lax.broadcasted_iota(jnp.int32, (1, m), 1)
    d0 = (xs - x0) ** 2 + (ys - y0) ** 2 + (zs - z0) ** 2
    cx_ref[...] = jnp.broadcast_to(x0, cx_ref.shape)
    cy_ref[...] = jnp.broadcast_to(y0, cy_ref.shape)
    cz_ref[...] = jnp.broadcast_to(z0, cz_ref.shape)

    def body(i, dists):
        mx = jnp.max(dists, axis=1, keepdims=True)
        sel = jnp.min(jnp.where(dists == mx, iota, n), axis=1, keepdims=True)
        mask = (iota == sel).astype(_F32)
        xc = jnp.sum(xs * mask, axis=1, keepdims=True)
        yc = jnp.sum(ys * mask, axis=1, keepdims=True)
        zc = jnp.sum(zs * mask, axis=1, keepdims=True)
        dnew = (xs - xc) ** 2 + (ys - yc) ** 2 + (zs - zc) ** 2
        hit = lane_m == i
        cx_ref[...] = jnp.where(hit, xc, cx_ref[...])
        cy_ref[...] = jnp.where(hit, yc, cy_ref[...])
        cz_ref[...] = jnp.where(hit, zc, cz_ref[...])
        return jnp.minimum(dists, dnew)

    jax.lax.fori_loop(1, m, body, d0)


def _sa_kernel(featxyz_ref, cen_ref, xyzT_ref,
               w1_ref, b1_ref, w2_ref, b2_ref, w3_ref, b3_ref,
               fout_ref, d_ref,
               *, n, m, k, cf):
    cen = cen_ref[...]
    sxyzT = xyzT_ref[...]
    D = ((cen[:, 0:1] - sxyzT[0:1, :]) ** 2
         + (cen[:, 1:2] - sxyzT[1:2, :]) ** 2)
    d_ref[...] = D + (cen[:, 2:3] - sxyzT[2:3, :]) ** 2

    lane_mn = jax.lax.broadcasted_iota(jnp.int32, (m, n), 1)
    cenpad = jnp.concatenate([jnp.zeros((m, cf), _F32), cen], axis=1)
    featxyz = featxyz_ref[...]
    w1, b1 = w1_ref[...], b1_ref[...]
    w2, b2 = w2_ref[...], b2_ref[...]
    w3, b3 = w3_ref[...], b3_ref[...]

    def nbr_body(_, acc):
        D = d_ref[...]
        oh, _, idx = _first_min_onehot(D, lane_mn, n)
        d_ref[...] = jnp.where(lane_mn == idx, _BIG, D)
        g = _dot(oh, featxyz) - cenpad
        a = jnp.maximum(_dot(g, w1) + b1, 0.0)
        a = jnp.maximum(_dot(a, w2) + b2, 0.0)
        a = jnp.maximum(_dot(a, w3) + b3, 0.0)
        return jnp.maximum(acc, a)

    cout = w3.shape[1]
    fout_ref[...] = jax.lax.fori_loop(
        0, k, nbr_body, jnp.zeros((m, cout), _F32))


def _sa(layers, feat, xyz, m, k):
    G, n, cf = feat.shape
    (w1, b1), (w2, b2), (w3, b3) = layers
    cout = w3.shape[1]
    featxyz = jnp.concatenate([feat, xyz], axis=2)
    xyzT = jnp.transpose(xyz, (0, 2, 1))
    cx, cy, cz = pl.pallas_call(
        functools.partial(_fps_kernel, n=n, m=m),
        out_shape=(jax.ShapeDtypeStruct((G, m), _F32),) * 3,
    )(xyzT[:, 0], xyzT[:, 1], xyzT[:, 2])
    cen = jnp.stack([cx, cy, cz], axis=2)
    fn = pl.pallas_call(
        functools.partial(_sa_kernel, n=n, m=m, k=k, cf=cf),
        out_shape=jax.ShapeDtypeStruct((m, cout), _F32),
        scratch_shapes=[pltpu.VMEM((m, n), _F32)],
    )
    fout = jax.vmap(fn, in_axes=(0, 0, 0) + (None,) * 6)(
        featxyz, cen, xyzT,
        w1, b1.reshape(1, -1), w2, b2.reshape(1, -1), w3, b3.reshape(1, -1))
    return fout, cen


def _lpt_kernel(fcur_ref, fsrc_ref, qxyz_ref, sxyzT_ref,
                wq_ref, wk_ref, wv_ref, out_ref, d_ref, s_ref, i_ref,
                *, m, n, k, c):
    qxyz = qxyz_ref[...]
    sxyzT = sxyzT_ref[...]
    D = ((qxyz[:, 0:1] - sxyzT[0:1, :]) ** 2
         + (qxyz[:, 1:2] - sxyzT[1:2, :]) ** 2)
    d_ref[...] = D + (qxyz[:, 2:3] - sxyzT[2:3, :]) ** 2
    lane_mn = jax.lax.broadcasted_iota(jnp.int32, (m, n), 1)

    q = _dot(fcur_ref[...], wq_ref[...])
    K = _dot(fsrc_ref[...], wk_ref[...])
    V = _dot(fsrc_ref[...], wv_ref[...])
    S = jax.lax.dot_general(q, K, (((1,), (1,)), ((), ())),
                            preferred_element_type=_F32,
                            precision=_PREC)

    def sel_body(j, _):
        D = d_ref[...]
        oh, _, idx = _first_min_onehot(D, lane_mn, n)
        d_ref[...] = jnp.where(lane_mn == idx, _BIG, D)
        s_ref[j] = jnp.sum(S * oh, axis=1, keepdims=True)
        i_ref[j] = idx
        return 0

    jax.lax.fori_loop(0, k, sel_body, 0)

    s = s_ref[...] / np.sqrt(c)
    e = jnp.exp(s - jnp.max(s, axis=0, keepdims=True))
    s_ref[...] = e / jnp.sum(e, axis=0, keepdims=True)

    def mix_body(j, A):
        return A + s_ref[j] * (lane_mn == i_ref[j]).astype(_F32)

    A = jax.lax.fori_loop(0, k, mix_body, jnp.zeros((m, n), _F32))
    out_ref[...] = _dot(A, V)


def _lpt(p, f_cur, f_src, xyz_cur, xyz_src, k):
    B, m, c = f_cur.shape
    n = f_src.shape[1]
    sxyzT = jnp.transpose(xyz_src, (0, 2, 1))
    fn = pl.pallas_call(
        functools.partial(_lpt_kernel, m=m, n=n, k=k, c=c),
        out_shape=jax.ShapeDtypeStruct((m, c), _F32),
        scratch_shapes=[pltpu.VMEM((m, n), _F32),
                        pltpu.VMEM((k, m, 1), _F32),
                        pltpu.VMEM((k, m, 1), jnp.int32)],
    )
    return jax.vmap(fn, in_axes=(0, 0, 0, 0, None, None, None))(
        f_cur, f_src, xyz_cur, sxyzT, p['Wq'], p['Wk'], p['Wv'])


def _lstm3_kernel(*refs):
    for lvl in range(3):
        fb_ref, ff_ref, h_ref, c_ref = refs[4 * lvl:4 * lvl + 4]
        wx_ref, wh_ref, b_ref = refs[12 + 3 * lvl:15 + 3 * lvl]
        hout_ref, cout_ref = refs[21 + 2 * lvl:23 + 2 * lvl]
        hdim = h_ref.shape[1]
        x = jnp.concatenate([fb_ref[...], ff_ref[...]], axis=1)
        g = _dot(x, wx_ref[...]) + _dot(h_ref[...], wh_ref[...]) + b_ref[...]
        i = jax.nn.sigmoid(g[:, 0:hdim])
        f = jax.nn.sigmoid(g[:, hdim:2 * hdim])
        gg = jnp.tanh(g[:, 2 * hdim:3 * hdim])
        o = jax.nn.sigmoid(g[:, 3 * hdim:4 * hdim])
        cn = f * c_ref[...] + i * gg
        hout_ref[...] = o * jnp.tanh(cn)
        cout_ref[...] = cn


def _lstm3(p, st, fbff):
    H1, C1, H2, C2, H3, C3 = st
    fb1, ff1, fb2, ff2, fb3, ff3 = fbff
    B = H1.shape[0]
    shapes = tuple(jax.ShapeDtypeStruct(h.shape[1:], _F32)
                   for h in (H1, H1, H2, H2, H3, H3))
    fn = pl.pallas_call(_lstm3_kernel, out_shape=shapes)
    ws = []
    for name in ('lstm1', 'lstm2', 'lstm3'):
        ws += [p[name]['Wx'], p[name]['Wh'], p[name]['b'].reshape(1, -1)]
    return jax.vmap(fn, in_axes=(0,) * 12 + (None,) * 9)(
        fb1, ff1, H1, C1, fb2, ff2, H2, C2, fb3, ff3, H3, C3, *ws)


def _interp(xc, posf, poscT, d_ref, m, n, k):
    D = ((posf[:, 0:1] - poscT[0:1, :]) ** 2
         + (posf[:, 1:2] - poscT[1:2, :]) ** 2)
    d_ref[...] = D + (posf[:, 2:3] - poscT[2:3, :]) ** 2
    lane_mn = jax.lax.broadcasted_iota(jnp.int32, (m, n), 1)

    def body(_, carry):
        W, wsum = carry
        D = d_ref[...]
        oh, mn, idx = _first_min_onehot(D, lane_mn, n)
        d_ref[...] = jnp.where(lane_mn == idx, _BIG, D)
        w = 1.0 / (mn + 1e-2)
        return W + w * oh, wsum + w

    W, wsum = jax.lax.fori_loop(
        0, k, body, (jnp.zeros((m, n), _F32), jnp.zeros((m, 1), _F32)))
    return _dot(W, xc) / wsum


def _fp_kernel(xc_ref, posf_ref, poscT_ref, xskip_ref,
               w1_ref, b1_ref, w2_ref, b2_ref, out_ref, d_ref, *, m, n, k):
    interp = _interp(xc_ref[...], posf_ref[...], poscT_ref[...], d_ref, m, n, k)
    h = jnp.concatenate([interp, xskip_ref[...]], axis=1)
    h = jnp.maximum(_dot(h, w1_ref[...]) + b1_ref[...], 0.0)
    h = jnp.maximum(_dot(h, w2_ref[...]) + b2_ref[...], 0.0)
    out_ref[...] = h


def _fp(layers, x_c, pos_c, x_skip, pos_f, k):
    B, m, _ = pos_f.shape
    n = pos_c.shape[1]
    (w1, b1), (w2, b2) = layers
    poscT = jnp.transpose(pos_c, (0, 2, 1))
    fn = pl.pallas_call(
        functools.partial(_fp_kernel, m=m, n=n, k=k),
        out_shape=jax.ShapeDtypeStruct((m, w2.shape[1]), _F32),
        scratch_shapes=[pltpu.VMEM((m, n), _F32)],
    )
    return jax.vmap(fn, in_axes=(0, 0, 0, 0, None, None, None, None))(
        x_c, pos_f, poscT, x_skip, w1, b1.reshape(1, -1), w2, b2.reshape(1, -1))


def _fpns_cls_kernel(xc_ref, posf_ref, poscT_ref,
                     w1_ref, b1_ref, w2_ref, b2_ref,
                     c1_ref, c2_ref, c3_ref, c4_ref, out_ref, d_ref,
                     *, m, n, k):
    interp = _interp(xc_ref[...], posf_ref[...], poscT_ref[...], d_ref, m, n, k)
    h = jnp.maximum(_dot(interp, w1_ref[...]) + b1_ref[...], 0.0)
    h = jnp.maximum(_dot(h, w2_ref[...]) + b2_ref[...], 0.0)
    h = _dot(h, c1_ref[...])
    h = _dot(h, c2_ref[...])
    h = _dot(h, c3_ref[...])
    h = _dot(h, c4_ref[...])
    out_ref[...] = posf_ref[...] + h


def _fpns_cls(layers, cls, x_c, pos_c, pos_f, k):
    B, m, _ = pos_f.shape
    n = pos_c.shape[1]
    (w1, b1), (w2, b2) = layers
    c1, c2, c3, c4 = cls
    poscT = jnp.transpose(pos_c, (0, 2, 1))
    fn = pl.pallas_call(
        functools.partial(_fpns_cls_kernel, m=m, n=n, k=k),
        out_shape=jax.ShapeDtypeStruct((m, 3), _F32),
        scratch_shapes=[pltpu.VMEM((m, n), _F32)],
    )
    return jax.vmap(fn, in_axes=(0, 0, 0) + (None,) * 8)(
        x_c, pos_f, poscT, w1, b1.reshape(1, -1), w2, b2.reshape(1, -1),
        c1, c2, c3, c4)


def kernel(input_xyz, num_pred, params):
    p = params
    T, B, _, N = input_xyz.shape
    frames = jnp.transpose(input_xyz, (0, 1, 3, 2))
    N1, N2, N3 = N // 16, N // 32, N // 64

    def encode(fr):
        f1, x1 = _sa(p['sa1'], fr, fr, N1, 32)
        f2, x2 = _sa(p['sa2'], f1, x1, N2, 16)
        f3, x3 = _sa(p['sa3'], f2, x2, N3, 8)
        return (f1, x1, f2, x2, f3, x3)

    e_all = encode(frames.reshape(T * B, N, 3))
    encs = [tuple(a.reshape((T, B) + a.shape[1:])[t] for a in e_all)
            for t in range(T)]

    st = (jnp.zeros((B, N1, 128), _F32), jnp.zeros((B, N1, 128), _F32),
          jnp.zeros((B, N2, 256), _F32), jnp.zeros((B, N2, 256), _F32),
          jnp.zeros((B, N3, 512), _F32), jnp.zeros((B, N3, 512), _F32))

    def lpt_all(pairs, p_l, k_l, fi, pi):
        f_cur = jnp.concatenate([c[fi] for c, _ in pairs], axis=0)
        f_src = jnp.concatenate([s[fi] for _, s in pairs], axis=0)
        x_cur = jnp.concatenate([c[pi] for c, _ in pairs], axis=0)
        x_src = jnp.concatenate([s[pi] for _, s in pairs], axis=0)
        out = _lpt(p_l, f_cur, f_src, x_cur, x_src, k_l)
        return out.reshape((len(pairs), B) + out.shape[1:])

    def lpt_levels(pairs):
        a1 = lpt_all(pairs, p['gat1'], 16, 0, 1)
        a2 = lpt_all(pairs, p['gat2'], 16, 2, 3)
        a3 = lpt_all(pairs, p['gat3'], 8, 4, 5)
        return a1, a2, a3

    pairs = []
    for t in range(T):
        prev = encs[t - 1] if t > 0 else encs[0]
        nxt = encs[t + 1] if t < T - 1 else encs[t]
        pairs += [(encs[t], prev), (encs[t], nxt)]
    a1, a2, a3 = lpt_levels(pairs)
    for t in range(T):
        st = _lstm3(p, st, (a1[2 * t], a1[2 * t + 1], a2[2 * t],
                            a2[2 * t + 1], a3[2 * t], a3[2 * t + 1]))

    def decode(st, e, fine_xyz):
        H1, _, H2, _, H3, _ = st
        x2 = _fp(p['fp32'], H3, e[5], H2, e[3], 8)
        x1 = _fp(p['fp21'], x2, e[3], H1, e[1], 16)
        return _fpns_cls(p['fp10'], p['cls'], x1, e[1], fine_xyz, 32)

    num_steps = 2
    pc_next = decode(st, encs[-1], frames[-1])
    preds = [pc_next]
    for _ in range(1, num_steps):
        e_new = encode(pc_next)
        b1, b2, b3 = lpt_levels([(e_new, encs[-1]), (e_new, e_new)])
        st = _lstm3(p, st, (b1[0], b1[1], b2[0], b2[1], b3[0], b3[1]))
        encs.append(e_new)
        pc_next = decode(st, e_new, pc_next)
        preds.append(pc_next)
    return jnp.stack(preds)
